# baseline (speedup 1.0000x reference)
import functools

import jax
import jax.numpy as jnp
from jax import lax
from jax.experimental import pallas as pl
from jax.experimental.pallas import tpu as pltpu
from jax.experimental.pallas import tpu_sc as plsc

N = 10000
D = 128
DH = D // 2
E = 320000
NC = 2
NS = 16
K = 80
EPT = E // NS
NCH = EPT // K
HBINS = 10240
HSTRIPE = HBINS // NS
APAD = 10240
ASTRIPE = APAD // NS
HEPT = E // (NC * NS)
HNCH = HEPT // K


@functools.cache
def _sc_kernels():
    mesh = plsc.VectorSubcoreMesh(core_axis_name="c", subcore_axis_name="s")

    @functools.partial(
        pl.kernel,
        mesh=mesh,
        compiler_params=pltpu.CompilerParams(use_tc_tiling_on_sc=False),
        out_type=jax.ShapeDtypeStruct((NC * HBINS,), jnp.float32),
        scratch_types=[
            pltpu.VMEM((HNCH, K), jnp.int32),
            pltpu.VMEM((HNCH, K), jnp.int32),
            pltpu.VMEM((K,), jnp.float32),
            pltpu.VMEM((HSTRIPE,), jnp.float32),
            pltpu.VMEM_SHARED((HBINS,), jnp.float32),
        ],
    )
    def _deg_kernel(epk, out, pkv, colv, onesv, zb, hist):
        cid = lax.axis_index("c")
        sid = lax.axis_index("s")
        pltpu.sync_copy(epk.at[sid, pl.ds(cid * HNCH, HNCH)], pkv)

        def _unprow(j, _):
            for g in range(K // 16):
                sl = pl.ds(g * 16, 16)
                colv[j, sl] = lax.shift_right_logical(pkv[j, sl], 14)
            return 0

        lax.fori_loop(0, HNCH, _unprow, 0)

        def _fill(i, _):
            onesv[pl.ds(i * 16, 16)] = jnp.ones((16,), jnp.float32)
            return 0

        lax.fori_loop(0, K // 16, _fill, 0)

        def _zero(i, _):
            zb[pl.ds(i * 16, 16)] = jnp.zeros((16,), jnp.float32)
            return 0

        lax.fori_loop(0, HSTRIPE // 16, _zero, 0)
        pltpu.sync_copy(zb, hist.at[pl.ds(sid * HSTRIPE, HSTRIPE)])
        plsc.subcore_barrier()

        def _add(j, _):
            pltpu.sync_copy(onesv, hist.at[colv.at[j]], add=True)
            return 0

        lax.fori_loop(0, HNCH, _add, 0)
        plsc.subcore_barrier()
        pltpu.sync_copy(hist.at[pl.ds(sid * HSTRIPE, HSTRIPE)],
                        out.at[pl.ds(cid * HBINS + sid * HSTRIPE, HSTRIPE)])

    @functools.partial(
        pl.kernel,
        mesh=mesh,
        compiler_params=pltpu.CompilerParams(use_tc_tiling_on_sc=False),
        out_type=jax.ShapeDtypeStruct((NC, APAD, DH), jnp.float32),
        scratch_types=[
            pltpu.VMEM((NCH, K), jnp.int32),
            pltpu.VMEM((NCH, K), jnp.int32),
            pltpu.VMEM((NCH, K), jnp.int32),
            pltpu.VMEM((K, DH), jnp.float32),
            pltpu.VMEM((K, DH), jnp.float32),
            pltpu.VMEM((128, DH), jnp.float32),
            pltpu.VMEM_SHARED((APAD, DH), jnp.float32),
            pltpu.SemaphoreType.DMA,
            pltpu.SemaphoreType.DMA,
            pltpu.SemaphoreType.DMA,
            pltpu.SemaphoreType.DMA,
        ],
    )
    def _agg_kernel(epk, ya, yb, out, pkv, rowv, colv, buf0, buf1, zb,
                    acc, sg0, sg1, ss0, ss1):
        cid = lax.axis_index("c")
        sid = lax.axis_index("s")
        pltpu.sync_copy(epk.at[sid], pkv)

        def _unprow(j, _):
            for g in range(K // 16):
                sl = pl.ds(g * 16, 16)
                p = pkv[j, sl]
                rowv[j, sl] = p & 0x3FFF
                colv[j, sl] = lax.shift_right_logical(p, 14)
            return 0

        lax.fori_loop(0, NCH, _unprow, 0)

        def _zero(i, _):
            r = i // 4
            c = (i % 4) * 16
            zb[r, pl.ds(c, 16)] = jnp.zeros((16,), jnp.float32)
            return 0

        lax.fori_loop(0, 128 * 4, _zero, 0)
        for m in range(ASTRIPE // 128):
            pltpu.sync_copy(zb, acc.at[pl.ds(sid * ASTRIPE + m * 128, 128)])
        plsc.subcore_barrier()

        def _sweep(src):
            def _wait_g(buf, sem):
                pltpu.make_async_copy(src.at[rowv.at[0]], buf, sem).wait()

            def _wait_s(buf, sem):
                pltpu.make_async_copy(buf, acc.at[colv.at[0]], sem).wait()

            pltpu.async_copy(src.at[rowv.at[0]], buf0, sg0)
            pltpu.async_copy(src.at[rowv.at[1]], buf1, sg1)

            def _pair(t, _):
                j0 = 2 * t
                _wait_g(buf0, sg0)
                pltpu.async_copy(buf0, acc.at[colv.at[j0]], ss0, add=True)
                _wait_g(buf1, sg1)
                pltpu.async_copy(buf1, acc.at[colv.at[j0 + 1]], ss1, add=True)

                @pl.when(j0 + 2 < NCH)
                def _():
                    _wait_s(buf0, ss0)
                    pltpu.async_copy(src.at[rowv.at[j0 + 2]], buf0, sg0)

                @pl.when(j0 + 3 < NCH)
                def _():
                    _wait_s(buf1, ss1)
                    pltpu.async_copy(src.at[rowv.at[j0 + 3]], buf1, sg1)

                return 0

            lax.fori_loop(0, NCH // 2, _pair, 0)
            _wait_s(buf0, ss0)
            _wait_s(buf1, ss1)

        @pl.when(cid == 0)
        def _():
            _sweep(ya)

        @pl.when(cid == 1)
        def _():
            _sweep(yb)

        plsc.subcore_barrier()
        pltpu.sync_copy(acc.at[pl.ds(sid * ASTRIPE, ASTRIPE)],
                        out.at[cid, pl.ds(sid * ASTRIPE, ASTRIPE)])

    return _deg_kernel, _agg_kernel


RB = 1000


def _dot_t(a, w_ref):
    return lax.dot_general(a, w_ref[...], (((1,), (1,)), ((), ())),
                           preferred_element_type=jnp.float32)


def _lin_body(x_ref, wt_ref, dga_ref, dgb_ref, ya_ref, yb_ref, dinv_ref):
    xl = _dot_t(x_ref[...], wt_ref)
    deg = dga_ref[...] + dgb_ref[...] + 1.0
    dinv = lax.rsqrt(deg)
    y = dinv * xl
    ya_ref[...] = y[:, :DH]
    yb_ref[...] = y[:, DH:]
    dinv_ref[...] = dinv


def _dense_body(acca_ref, accb_ref, ya_ref, yb_ref, dinv_ref, bg_ref,
                wu1_ref, bu1_ref, wu2_ref, bu2_ref,
                wl1_ref, bl1_ref, wl2_ref, bl2_ref,
                wlu_ref, wll_ref, blast_ref, out_ref):
    dinv = dinv_ref[...]
    sl = dinv * (acca_ref[0] + ya_ref[...])
    sr = dinv * (accb_ref[0] + yb_ref[...])
    s = jnp.concatenate([sl, sr], axis=1) + bg_ref[...]
    fst = jnp.maximum(s, 0.0)
    u = _dot_t(fst, wu1_ref) + bu1_ref[...]
    u = _dot_t(u, wu2_ref) + bu2_ref[...]
    v = _dot_t(fst, wl1_ref) + bl1_ref[...]
    v = _dot_t(v, wl2_ref) + bl2_ref[...]
    last = _dot_t(u, wlu_ref) + _dot_t(v, wll_ref) + blast_ref[...]
    out_ref[...] = jnp.maximum(last, 0.0)


def _row_spec(w):
    return pl.BlockSpec((RB, w), lambda i: (i, 0))


def _full_spec(h, w):
    return pl.BlockSpec((h, w), lambda i: (0, 0))


def kernel(x, edge_index, W_lin, b_gcn, Wu1, bu1, Wu2, bu2,
           Wl1, bl1, Wl2, bl2, Wlast, blast):
    ei32 = edge_index.astype(jnp.int32)
    epk = (ei32[0] | (ei32[1] << 14)).reshape(NS, NCH, K)

    _deg_kernel, _agg_kernel = _sc_kernels()
    deg2 = _deg_kernel(epk)
    dga = deg2[:N].reshape(N, 1)
    dgb = deg2[HBINS:HBINS + N].reshape(N, 1)

    ya, yb, dinv = pl.pallas_call(
        _lin_body,
        grid=(N // RB,),
        in_specs=[_row_spec(D), _full_spec(D, D), _row_spec(1), _row_spec(1)],
        out_specs=[_row_spec(DH), _row_spec(DH), _row_spec(1)],
        out_shape=[jax.ShapeDtypeStruct((N, DH), jnp.float32),
                   jax.ShapeDtypeStruct((N, DH), jnp.float32),
                   jax.ShapeDtypeStruct((N, 1), jnp.float32)],
    )(x, W_lin, dga, dgb)

    acc2 = _agg_kernel(epk, ya, yb)

    out = pl.pallas_call(
        _dense_body,
        grid=(N // RB,),
        in_specs=[pl.BlockSpec((1, RB, DH), lambda i: (0, i, 0)),
                  pl.BlockSpec((1, RB, DH), lambda i: (1, i, 0)),
                  _row_spec(DH), _row_spec(DH),
                  _row_spec(1),
                  _full_spec(1, D),
                  _full_spec(D, D), _full_spec(1, D),
                  _full_spec(D, D), _full_spec(1, D),
                  _full_spec(D, D), _full_spec(1, D),
                  _full_spec(D, D), _full_spec(1, D),
                  _full_spec(2 * D, D), _full_spec(2 * D, D),
                  _full_spec(1, 2 * D)],
        out_specs=_row_spec(2 * D),
        out_shape=jax.ShapeDtypeStruct((N, 2 * D), jnp.float32),
    )(acc2, acc2, ya, yb, dinv, b_gcn.reshape(1, D),
      Wu1, bu1.reshape(1, D), Wu2, bu2.reshape(1, D),
      Wl1, bl1.reshape(1, D), Wl2, bl2.reshape(1, D),
      Wlast[:, :D], Wlast[:, D:], blast.reshape(1, 2 * D))
    return out


# speedup vs baseline: 1.2960x; 1.2960x over previous
import functools

import jax
import jax.numpy as jnp
from jax import lax
from jax.experimental import pallas as pl
from jax.experimental.pallas import tpu as pltpu
from jax.experimental.pallas import tpu_sc as plsc

N = 10000
D = 128
DH = D // 2
E = 320000
NC = 2
NS = 16
K = 125
EPT = E // NS
NCH = EPT // K
HBINS = 10240
HSTRIPE = HBINS // NS
APAD = 10240
ASTRIPE = APAD // NS
HEPT = E // (NC * NS)
HNCH = HEPT // K


@functools.cache
def _sc_kernels():
    mesh = plsc.VectorSubcoreMesh(core_axis_name="c", subcore_axis_name="s")

    @functools.partial(
        pl.kernel,
        mesh=mesh,
        compiler_params=pltpu.CompilerParams(use_tc_tiling_on_sc=False),
        out_type=jax.ShapeDtypeStruct((NC * HBINS,), jnp.float32),
        scratch_types=[
            pltpu.VMEM((HNCH, K), jnp.int32),
            pltpu.VMEM((128,), jnp.float32),
            pltpu.VMEM((HSTRIPE,), jnp.float32),
            pltpu.VMEM_SHARED((HBINS,), jnp.float32),
        ],
    )
    def _deg_kernel(edge4, out, colv, onesv, zb, hist):
        cid = lax.axis_index("c")
        sid = lax.axis_index("s")
        pltpu.sync_copy(edge4.at[1, sid, pl.ds(cid * HNCH, HNCH)], colv)

        def _fill(i, _):
            onesv[pl.ds(i * 16, 16)] = jnp.ones((16,), jnp.float32)
            return 0

        lax.fori_loop(0, 8, _fill, 0)

        def _zero(i, _):
            zb[pl.ds(i * 16, 16)] = jnp.zeros((16,), jnp.float32)
            return 0

        lax.fori_loop(0, HSTRIPE // 16, _zero, 0)
        pltpu.sync_copy(zb, hist.at[pl.ds(sid * HSTRIPE, HSTRIPE)])
        plsc.subcore_barrier()

        def _add(j, _):
            pltpu.sync_copy(onesv.at[pl.ds(0, K)], hist.at[colv.at[j]], add=True)
            return 0

        lax.fori_loop(0, HNCH, _add, 0)
        plsc.subcore_barrier()
        pltpu.sync_copy(hist.at[pl.ds(sid * HSTRIPE, HSTRIPE)],
                        out.at[pl.ds(cid * HBINS + sid * HSTRIPE, HSTRIPE)])

    @functools.partial(
        pl.kernel,
        mesh=mesh,
        compiler_params=pltpu.CompilerParams(use_tc_tiling_on_sc=False),
        out_type=jax.ShapeDtypeStruct((NC, APAD, DH), jnp.float32),
        scratch_types=[
            pltpu.VMEM((NCH, K), jnp.int32),
            pltpu.VMEM((NCH, K), jnp.int32),
            pltpu.VMEM((K, DH), jnp.float32),
            pltpu.VMEM((K, DH), jnp.float32),
            pltpu.VMEM((128, DH), jnp.float32),
            pltpu.VMEM_SHARED((APAD, DH), jnp.float32),
            pltpu.SemaphoreType.DMA,
            pltpu.SemaphoreType.DMA,
        ],
    )
    def _agg_kernel(edge4, ya, yb, out, rowv, colv, buf0, buf1, zb,
                    acc, sg0, sg1):
        cid = lax.axis_index("c")
        sid = lax.axis_index("s")
        pltpu.sync_copy(edge4.at[0, sid], rowv)
        pltpu.sync_copy(edge4.at[1, sid], colv)

        def _zero(i, _):
            r = i // 4
            c = (i % 4) * 16
            zb[r, pl.ds(c, 16)] = jnp.zeros((16,), jnp.float32)
            return 0

        lax.fori_loop(0, 128 * 4, _zero, 0)
        for m in range(ASTRIPE // 128):
            pltpu.sync_copy(zb, acc.at[pl.ds(sid * ASTRIPE + m * 128, 128)])
        plsc.subcore_barrier()

        def _sweep(src):
            def _wait_g(buf, sem):
                pltpu.make_async_copy(src.at[rowv.at[0]], buf, sem).wait()

            pltpu.async_copy(src.at[rowv.at[0]], buf0, sg0)

            def _pair(t, _):
                j0 = 2 * t
                pltpu.async_copy(src.at[rowv.at[j0 + 1]], buf1, sg1)
                _wait_g(buf0, sg0)
                pltpu.sync_copy(buf0, acc.at[colv.at[j0]], add=True)

                @pl.when(j0 + 2 < NCH)
                def _():
                    pltpu.async_copy(src.at[rowv.at[j0 + 2]], buf0, sg0)

                _wait_g(buf1, sg1)
                pltpu.sync_copy(buf1, acc.at[colv.at[j0 + 1]], add=True)
                return 0

            lax.fori_loop(0, NCH // 2, _pair, 0)

        @pl.when(cid == 0)
        def _():
            _sweep(ya)

        @pl.when(cid == 1)
        def _():
            _sweep(yb)

        plsc.subcore_barrier()
        pltpu.sync_copy(acc.at[pl.ds(sid * ASTRIPE, ASTRIPE)],
                        out.at[cid, pl.ds(sid * ASTRIPE, ASTRIPE)])

    return _deg_kernel, _agg_kernel


RB = 1000


def _dot_t(a, w_ref):
    return lax.dot_general(a, w_ref[...], (((1,), (1,)), ((), ())),
                           preferred_element_type=jnp.float32)


def _lin_body(x_ref, wt_ref, dga_ref, dgb_ref, ya_ref, yb_ref, dinv_ref):
    xl = _dot_t(x_ref[...], wt_ref)
    deg = dga_ref[...] + dgb_ref[...] + 1.0
    dinv = lax.rsqrt(deg)
    y = dinv * xl
    ya_ref[...] = y[:, :DH]
    yb_ref[...] = y[:, DH:]
    dinv_ref[...] = dinv


def _dense_body(acca_ref, accb_ref, ya_ref, yb_ref, dinv_ref, bg_ref,
                wu1_ref, bu1_ref, wu2_ref, bu2_ref,
                wl1_ref, bl1_ref, wl2_ref, bl2_ref,
                wlu_ref, wll_ref, blast_ref, out_ref):
    dinv = dinv_ref[...]
    sl = dinv * (acca_ref[0] + ya_ref[...])
    sr = dinv * (accb_ref[0] + yb_ref[...])
    s = jnp.concatenate([sl, sr], axis=1) + bg_ref[...]
    fst = jnp.maximum(s, 0.0)
    u = _dot_t(fst, wu1_ref) + bu1_ref[...]
    u = _dot_t(u, wu2_ref) + bu2_ref[...]
    v = _dot_t(fst, wl1_ref) + bl1_ref[...]
    v = _dot_t(v, wl2_ref) + bl2_ref[...]
    last = _dot_t(u, wlu_ref) + _dot_t(v, wll_ref) + blast_ref[...]
    out_ref[...] = jnp.maximum(last, 0.0)


def _row_spec(w):
    return pl.BlockSpec((RB, w), lambda i: (i, 0))


def _full_spec(h, w):
    return pl.BlockSpec((h, w), lambda i: (0, 0))


def kernel(x, edge_index, W_lin, b_gcn, Wu1, bu1, Wu2, bu2,
           Wl1, bl1, Wl2, bl2, Wlast, blast):
    edge4 = edge_index.astype(jnp.int32).reshape(2, NS, NCH, K)

    _deg_kernel, _agg_kernel = _sc_kernels()
    deg2 = _deg_kernel(edge4)
    dga = deg2[:N].reshape(N, 1)
    dgb = deg2[HBINS:HBINS + N].reshape(N, 1)

    ya, yb, dinv = pl.pallas_call(
        _lin_body,
        grid=(N // RB,),
        in_specs=[_row_spec(D), _full_spec(D, D), _row_spec(1), _row_spec(1)],
        out_specs=[_row_spec(DH), _row_spec(DH), _row_spec(1)],
        out_shape=[jax.ShapeDtypeStruct((N, DH), jnp.float32),
                   jax.ShapeDtypeStruct((N, DH), jnp.float32),
                   jax.ShapeDtypeStruct((N, 1), jnp.float32)],
    )(x, W_lin, dga, dgb)

    acc2 = _agg_kernel(edge4, ya, yb)

    out = pl.pallas_call(
        _dense_body,
        grid=(N // RB,),
        in_specs=[pl.BlockSpec((1, RB, DH), lambda i: (0, i, 0)),
                  pl.BlockSpec((1, RB, DH), lambda i: (1, i, 0)),
                  _row_spec(DH), _row_spec(DH),
                  _row_spec(1),
                  _full_spec(1, D),
                  _full_spec(D, D), _full_spec(1, D),
                  _full_spec(D, D), _full_spec(1, D),
                  _full_spec(D, D), _full_spec(1, D),
                  _full_spec(D, D), _full_spec(1, D),
                  _full_spec(2 * D, D), _full_spec(2 * D, D),
                  _full_spec(1, 2 * D)],
        out_specs=_row_spec(2 * D),
        out_shape=jax.ShapeDtypeStruct((N, 2 * D), jnp.float32),
    )(acc2, acc2, ya, yb, dinv, b_gcn.reshape(1, D),
      Wu1, bu1.reshape(1, D), Wu2, bu2.reshape(1, D),
      Wl1, bl1.reshape(1, D), Wl2, bl2.reshape(1, D),
      Wlast[:, :D], Wlast[:, D:], blast.reshape(1, 2 * D))
    return out


# speedup vs baseline: 1.2961x; 1.0001x over previous
import functools

import jax
import jax.numpy as jnp
from jax import lax
from jax.experimental import pallas as pl
from jax.experimental.pallas import tpu as pltpu
from jax.experimental.pallas import tpu_sc as plsc

N = 10000
D = 128
DH = D // 2
E = 320000
NC = 2
NS = 16
K = 125
EPT = E // NS
NCH = EPT // K
HBINS = 10240
HSTRIPE = HBINS // NS
APAD = 10240
ASTRIPE = APAD // NS
HEPT = E // (NC * NS)
HNCH = HEPT // K


@functools.cache
def _sc_kernels():
    mesh = plsc.VectorSubcoreMesh(core_axis_name="c", subcore_axis_name="s")

    @functools.partial(
        pl.kernel,
        mesh=mesh,
        compiler_params=pltpu.CompilerParams(use_tc_tiling_on_sc=False),
        out_type=jax.ShapeDtypeStruct((NC * HBINS,), jnp.float32),
        scratch_types=[
            pltpu.VMEM((HNCH, K), jnp.int32),
            pltpu.VMEM((128,), jnp.float32),
            pltpu.VMEM((HSTRIPE,), jnp.float32),
            pltpu.VMEM_SHARED((HBINS,), jnp.float32),
        ],
    )
    def _deg_kernel(edge4, out, colv, onesv, zb, hist):
        cid = lax.axis_index("c")
        sid = lax.axis_index("s")
        pltpu.sync_copy(edge4.at[1, sid, pl.ds(cid * HNCH, HNCH)], colv)

        def _fill(i, _):
            onesv[pl.ds(i * 16, 16)] = jnp.ones((16,), jnp.float32)
            return 0

        lax.fori_loop(0, 8, _fill, 0)

        def _zero(i, _):
            zb[pl.ds(i * 16, 16)] = jnp.zeros((16,), jnp.float32)
            return 0

        lax.fori_loop(0, HSTRIPE // 16, _zero, 0)
        pltpu.sync_copy(zb, hist.at[pl.ds(sid * HSTRIPE, HSTRIPE)])
        plsc.subcore_barrier()

        def _add(j, _):
            pltpu.sync_copy(onesv.at[pl.ds(0, K)], hist.at[colv.at[j]], add=True)
            return 0

        lax.fori_loop(0, HNCH, _add, 0)
        plsc.subcore_barrier()
        pltpu.sync_copy(hist.at[pl.ds(sid * HSTRIPE, HSTRIPE)],
                        out.at[pl.ds(cid * HBINS + sid * HSTRIPE, HSTRIPE)])

    @functools.partial(
        pl.kernel,
        mesh=mesh,
        compiler_params=pltpu.CompilerParams(use_tc_tiling_on_sc=False),
        out_type=jax.ShapeDtypeStruct((NC, APAD, DH), jnp.float32),
        scratch_types=[
            pltpu.VMEM((NCH, K), jnp.int32),
            pltpu.VMEM((NCH, K), jnp.int32),
            pltpu.VMEM((K, DH), jnp.float32),
            pltpu.VMEM((K, DH), jnp.float32),
            pltpu.VMEM((128, DH), jnp.float32),
            pltpu.VMEM_SHARED((APAD, DH), jnp.float32),
            pltpu.SemaphoreType.DMA,
            pltpu.SemaphoreType.DMA,
        ],
    )
    def _agg_kernel(edge4, ya, yb, out, rowv, colv, buf0, buf1, zb,
                    acc, sg0, sg1):
        cid = lax.axis_index("c")
        sid = lax.axis_index("s")
        pltpu.sync_copy(edge4.at[0, sid], rowv)
        pltpu.sync_copy(edge4.at[1, sid], colv)

        def _zero(i, _):
            r = i // 4
            c = (i % 4) * 16
            zb[r, pl.ds(c, 16)] = jnp.zeros((16,), jnp.float32)
            return 0

        lax.fori_loop(0, 128 * 4, _zero, 0)
        for m in range(ASTRIPE // 128):
            pltpu.sync_copy(zb, acc.at[pl.ds(sid * ASTRIPE + m * 128, 128)])
        plsc.subcore_barrier()

        def _sweep(src):
            def _wait_g(buf, sem):
                pltpu.make_async_copy(src.at[rowv.at[0]], buf, sem).wait()

            pltpu.async_copy(src.at[rowv.at[0]], buf0, sg0)

            def _pair(t, _):
                j0 = 2 * t
                pltpu.async_copy(src.at[rowv.at[j0 + 1]], buf1, sg1)
                _wait_g(buf0, sg0)
                pltpu.sync_copy(buf0, acc.at[colv.at[j0]], add=True)

                @pl.when(j0 + 2 < NCH)
                def _():
                    pltpu.async_copy(src.at[rowv.at[j0 + 2]], buf0, sg0)

                _wait_g(buf1, sg1)
                pltpu.sync_copy(buf1, acc.at[colv.at[j0 + 1]], add=True)
                return 0

            lax.fori_loop(0, NCH // 2, _pair, 0)

        @pl.when(cid == 0)
        def _():
            _sweep(ya)

        @pl.when(cid == 1)
        def _():
            _sweep(yb)

        plsc.subcore_barrier()
        pltpu.sync_copy(acc.at[pl.ds(sid * ASTRIPE, ASTRIPE)],
                        out.at[cid, pl.ds(sid * ASTRIPE, ASTRIPE)])

    return _deg_kernel, _agg_kernel


RB = 1000


def _dot_t(a, w_ref):
    return lax.dot_general(a, w_ref[...], (((1,), (1,)), ((), ())),
                           preferred_element_type=jnp.float32)


def _xl_body(x_ref, wt_ref, xl_ref):
    xl_ref[...] = _dot_t(x_ref[...], wt_ref)


def _lin_body(xl_ref, dga_ref, dgb_ref, ya_ref, yb_ref, dinv_ref):
    deg = dga_ref[...] + dgb_ref[...] + 1.0
    dinv = lax.rsqrt(deg)
    y = dinv * xl_ref[...]
    ya_ref[...] = y[:, :DH]
    yb_ref[...] = y[:, DH:]
    dinv_ref[...] = dinv


def _dense_body(acca_ref, accb_ref, ya_ref, yb_ref, dinv_ref, bg_ref,
                wu1_ref, bu1_ref, wu2_ref, bu2_ref,
                wl1_ref, bl1_ref, wl2_ref, bl2_ref,
                wlu_ref, wll_ref, blast_ref, out_ref):
    dinv = dinv_ref[...]
    sl = dinv * (acca_ref[0] + ya_ref[...])
    sr = dinv * (accb_ref[0] + yb_ref[...])
    s = jnp.concatenate([sl, sr], axis=1) + bg_ref[...]
    fst = jnp.maximum(s, 0.0)
    u = _dot_t(fst, wu1_ref) + bu1_ref[...]
    u = _dot_t(u, wu2_ref) + bu2_ref[...]
    v = _dot_t(fst, wl1_ref) + bl1_ref[...]
    v = _dot_t(v, wl2_ref) + bl2_ref[...]
    last = _dot_t(u, wlu_ref) + _dot_t(v, wll_ref) + blast_ref[...]
    out_ref[...] = jnp.maximum(last, 0.0)


def _row_spec(w):
    return pl.BlockSpec((RB, w), lambda i: (i, 0))


def _full_spec(h, w):
    return pl.BlockSpec((h, w), lambda i: (0, 0))


def kernel(x, edge_index, W_lin, b_gcn, Wu1, bu1, Wu2, bu2,
           Wl1, bl1, Wl2, bl2, Wlast, blast):
    edge4 = edge_index.astype(jnp.int32).reshape(2, NS, NCH, K)

    _deg_kernel, _agg_kernel = _sc_kernels()
    deg2 = _deg_kernel(edge4)
    dga = deg2[:N].reshape(N, 1)
    dgb = deg2[HBINS:HBINS + N].reshape(N, 1)

    xl = pl.pallas_call(
        _xl_body,
        grid=(N // RB,),
        in_specs=[_row_spec(D), _full_spec(D, D)],
        out_specs=_row_spec(D),
        out_shape=jax.ShapeDtypeStruct((N, D), jnp.float32),
    )(x, W_lin)

    ya, yb, dinv = pl.pallas_call(
        _lin_body,
        grid=(N // RB,),
        in_specs=[_row_spec(D), _row_spec(1), _row_spec(1)],
        out_specs=[_row_spec(DH), _row_spec(DH), _row_spec(1)],
        out_shape=[jax.ShapeDtypeStruct((N, DH), jnp.float32),
                   jax.ShapeDtypeStruct((N, DH), jnp.float32),
                   jax.ShapeDtypeStruct((N, 1), jnp.float32)],
    )(xl, dga, dgb)

    acc2 = _agg_kernel(edge4, ya, yb)

    out = pl.pallas_call(
        _dense_body,
        grid=(N // RB,),
        in_specs=[pl.BlockSpec((1, RB, DH), lambda i: (0, i, 0)),
                  pl.BlockSpec((1, RB, DH), lambda i: (1, i, 0)),
                  _row_spec(DH), _row_spec(DH),
                  _row_spec(1),
                  _full_spec(1, D),
                  _full_spec(D, D), _full_spec(1, D),
                  _full_spec(D, D), _full_spec(1, D),
                  _full_spec(D, D), _full_spec(1, D),
                  _full_spec(D, D), _full_spec(1, D),
                  _full_spec(2 * D, D), _full_spec(2 * D, D),
                  _full_spec(1, 2 * D)],
        out_specs=_row_spec(2 * D),
        out_shape=jax.ShapeDtypeStruct((N, 2 * D), jnp.float32),
    )(acc2, acc2, ya, yb, dinv, b_gcn.reshape(1, D),
      Wu1, bu1.reshape(1, D), Wu2, bu2.reshape(1, D),
      Wl1, bl1.reshape(1, D), Wl2, bl2.reshape(1, D),
      Wlast[:, :D], Wlast[:, D:], blast.reshape(1, 2 * D))
    return out
